# baseline (speedup 1.0000x reference)
import functools

import jax
import jax.numpy as jnp
from jax import lax
from jax.experimental import pallas as pl
from jax.experimental.pallas import tpu as pltpu
from jax.experimental.pallas import tpu_sc as plsc

B = 32
P = 16
SG = 1000
NSAMP = 150
GRID = 32
EPS = 1e-12
BIG = 1e4

L = 16
SG_PAD = 1008
NS_PAD = 160
N_CH1 = SG_PAD // L
N_CH2 = NS_PAD // L
NCELL = GRID * GRID * GRID

f32 = jnp.float32
i32 = jnp.int32


def _rsqrt(x):
    i = plsc.bitcast(x, i32)
    y = plsc.bitcast(jnp.int32(0x5F3759DF) - lax.shift_right_logical(i, 1), f32)
    for _ in range(3):
        y = y * (1.5 - 0.5 * x * y * y)
    return y


def _sqrt(x):
    return x * _rsqrt(x)


def _cross(ax, ay, az, bx, by, bz):
    return ay * bz - az * by, az * bx - ax * bz, ax * by - ay * bx


def _rotate(qw, ux, uy, uz, vx, vy, vz):
    tx, ty, tz = _cross(ux, uy, uz, vx, vy, vz)
    tx, ty, tz = 2.0 * tx, 2.0 * ty, 2.0 * tz
    cx, cy, cz = _cross(ux, uy, uz, tx, ty, tz)
    return vx + qw * tx + cx, vy + qw * ty + cy, vz + qw * tz + cz


_MESH = plsc.VectorSubcoreMesh(
    core_axis_name="c", subcore_axis_name="s", num_cores=2, num_subcores=16
)


@functools.partial(
    pl.kernel,
    out_type=jax.ShapeDtypeStruct((B * 2 * L,), f32),
    mesh=_MESH,
    compiler_params=pltpu.CompilerParams(needs_layout_passes=False),
    scratch_types=[
        pltpu.VMEM((NCELL * 3,), f32),
        pltpu.VMEM((SG * 3,), f32),
        pltpu.VMEM((3 * SG_PAD,), f32),
        pltpu.VMEM((SG_PAD,), f32),
        pltpu.VMEM((3 * NS_PAD,), f32),
        pltpu.VMEM((P * 3,), f32),
        pltpu.VMEM((P * 3,), f32),
        pltpu.VMEM((P * 4,), f32),
        pltpu.VMEM((P,), i32),
        pltpu.VMEM((L,), f32),
        pltpu.VMEM((2 * L,), f32),
        pltpu.SemaphoreType.DMA,
    ],
)
def _sc_kernel(shape_hbm, trans_hbm, quat_hbm, cp_hbm, pts_hbm, inuse_hbm,
               surf_hbm, out_hbm, cp_v, pts_raw, pts_t, tsdf_v, surf_v,
               shape_v, trans_v, quat_v, inuse_v, acc2_v, out_v, cp_sem):
    b = lax.axis_index("s") * 2 + lax.axis_index("c")
    iota = jnp.arange(L, dtype=i32)

    cp_copy = pltpu.async_copy(
        cp_hbm.at[pl.ds(b * NCELL * 3, NCELL * 3)], cp_v, cp_sem)

    pltpu.sync_copy(pts_hbm.at[pl.ds(b * SG * 3, SG * 3)], pts_raw)
    pltpu.sync_copy(shape_hbm.at[pl.ds(b * P * 3, P * 3)], shape_v)
    pltpu.sync_copy(trans_hbm.at[pl.ds(b * P * 3, P * 3)], trans_v)
    pltpu.sync_copy(quat_hbm.at[pl.ds(b * P * 4, P * 4)], quat_v)
    pltpu.sync_copy(inuse_hbm.at[pl.ds(b * P, P)], inuse_v)
    pltpu.sync_copy(surf_hbm, surf_v)

    qw = plsc.load_gather(quat_v, [iota * 4])
    qx = plsc.load_gather(quat_v, [iota * 4 + 1])
    qy = plsc.load_gather(quat_v, [iota * 4 + 2])
    qz = plsc.load_gather(quat_v, [iota * 4 + 3])
    s = qw * qw + qx * qx + qy * qy + qz * qz
    n = s * _rsqrt(s)
    inv = 1.0 / (n + 1e-8)
    qnw, qnx, qny, qnz = qw * inv, qx * inv, qy * inv, qz * inv

    txv = plsc.load_gather(trans_v, [iota * 3])
    tyv = plsc.load_gather(trans_v, [iota * 3 + 1])
    tzv = plsc.load_gather(trans_v, [iota * 3 + 2])
    sxv = plsc.load_gather(shape_v, [iota * 3])
    syv = plsc.load_gather(shape_v, [iota * 3 + 1])
    szv = plsc.load_gather(shape_v, [iota * 3 + 2])
    iuv = inuse_v[:]

    big_vec = jnp.full((L,), BIG, f32)

    def stage_body(ci, carry):
        base = ci * L
        ridx = jnp.minimum(base + iota, SG - 1) * 3
        for comp in range(3):
            v = plsc.load_gather(pts_raw, [ridx + comp])
            pts_t[pl.ds(comp * SG_PAD + base, L)] = v
        tsdf_v[pl.ds(base, L)] = big_vec
        return carry

    lax.fori_loop(0, N_CH1, stage_body, 0)

    for p in range(P):
        iu = iuv[p]

        @pl.when(iu > 0)
        def _(p=p):
            w = qnw[p]
            nux = -qnx[p]
            nuy = -qny[p]
            nuz = -qnz[p]
            tx, ty, tz = txv[p], tyv[p], tzv[p]
            sx, sy, sz = sxv[p], syv[p], szv[p]

            def body(ci, carry):
                base = ci * L
                vx = pts_t[pl.ds(base, L)] - tx
                vy = pts_t[pl.ds(SG_PAD + base, L)] - ty
                vz = pts_t[pl.ds(2 * SG_PAD + base, L)] - tz
                lx, ly, lz = _rotate(w, nux, nuy, nuz, vx, vy, vz)
                dx = jnp.maximum(jnp.abs(lx) - sx, 0.0)
                dy = jnp.maximum(jnp.abs(ly) - sy, 0.0)
                dz = jnp.maximum(jnp.abs(lz) - sz, 0.0)
                t = dx * dx + dy * dy + dz * dz
                tsdf_v[pl.ds(base, L)] = jnp.minimum(tsdf_v[pl.ds(base, L)], t)
                return carry

            lax.fori_loop(0, N_CH1, body, 0)

    def red_body(ci, acc):
        v = tsdf_v[pl.ds(ci * L, L)] + EPS
        sq = _sqrt(v)
        valid = (ci * L + iota) < SG
        return acc + jnp.where(valid, sq, 0.0)

    acc1 = lax.fori_loop(0, N_CH1, red_body, jnp.zeros((L,), f32))

    cp_copy.wait()
    acc2_v[:] = jnp.zeros((L,), f32)
    sqrt_eps = _sqrt(jnp.full((L,), EPS, f32))
    onehot0 = jnp.where(iota == 0, 1.0, 0.0).astype(f32)

    for p in range(P):
        iu = iuv[p]

        @pl.when(iu > 0)
        def _(p=p):
            w = qnw[p]
            ux, uy, uz = qnx[p], qny[p], qnz[p]
            tx, ty, tz = txv[p], tyv[p], tzv[p]
            sx, sy, sz = sxv[p], syv[p], szv[p]

            def body(ci, acc):
                base = ci * L
                plx = surf_v[pl.ds(base, L)] * sx
                ply = surf_v[pl.ds(NS_PAD + base, L)] * sy
                plz = surf_v[pl.ds(2 * NS_PAD + base, L)] * sz
                px, py, pz = _rotate(w, ux, uy, uz, plx, ply, plz)
                px, py, pz = px + tx, py + ty, pz + tz
                gx = jnp.clip(((px + 0.5) * float(GRID)).astype(i32), 0, GRID - 1)
                gy = jnp.clip(((py + 0.5) * float(GRID)).astype(i32), 0, GRID - 1)
                gz = jnp.clip(((pz + 0.5) * float(GRID)).astype(i32), 0, GRID - 1)
                lin = ((gx * GRID + gy) * GRID + gz) * 3
                cx = plsc.load_gather(cp_v, [lin])
                cy = plsc.load_gather(cp_v, [lin + 1])
                cz = plsc.load_gather(cp_v, [lin + 2])
                ex, ey, ez = px - cx, py - cy, pz - cz
                d2 = ex * ex + ey * ey + ez * ez + EPS
                dist = _sqrt(d2)
                valid = (ci * L + iota) < NSAMP
                return acc + jnp.where(valid, dist, 0.0)

            acc_p = lax.fori_loop(0, N_CH2, body, jnp.zeros((L,), f32))
            acc2_v[:] = acc2_v[:] + acc_p

        @pl.when(iu <= 0)
        def _():
            acc2_v[:] = acc2_v[:] + onehot0 * (float(NSAMP) * sqrt_eps)

    out_v[pl.ds(0, L)] = acc1
    out_v[pl.ds(L, L)] = acc2_v[:]
    pltpu.sync_copy(out_v, out_hbm.at[pl.ds(b * 2 * L, 2 * L)])


def kernel(shape_rlt, trans_rlt, quat_rlt, CP, batchSamplepoint, inUse):
    tkey = jax.random.key(42)
    u = jax.random.uniform(tkey, (NSAMP, 3), dtype=f32, minval=-1.0, maxval=1.0)
    surf = u / jnp.max(jnp.abs(u), axis=-1, keepdims=True)
    surf_t = jnp.zeros((3, NS_PAD), f32).at[:, :NSAMP].set(surf.T).reshape(-1)

    out = _sc_kernel(
        shape_rlt.reshape(-1), trans_rlt.reshape(-1), quat_rlt.reshape(-1),
        CP.reshape(-1), batchSamplepoint.reshape(-1), inUse.reshape(-1),
        surf_t)
    out = out.reshape(B, 2, L)
    discd2 = jnp.sum(out[:, 0, :]) / (B * SG)
    discd1 = jnp.sum(out[:, 1, :]) / (B * P * NSAMP)
    return discd1 + discd2


# speedup vs baseline: 19.3503x; 19.3503x over previous
import functools

import numpy as np

import jax
import jax.numpy as jnp
from jax import lax
from jax.experimental import pallas as pl
from jax.experimental.pallas import tpu as pltpu
from jax.experimental.pallas import tpu_sc as plsc

B = 32
P = 16
SG = 1000
NSAMP = 150
GRID = 32
EPS = 1e-12
BIG = 1e4

L = 16
SG_PAD = 1008
NS_PAD = 160
U1 = 3
N_CH1 = SG_PAD // L
N_CH2 = NS_PAD // L
NCELL = GRID * GRID * GRID

f32 = jnp.float32
i32 = jnp.int32


def _surf_table():
    tkey = jax.random.key(42)
    u = jax.random.uniform(tkey, (NSAMP, 3), dtype=f32,
                           minval=-1.0, maxval=1.0)
    surf = u / jnp.max(jnp.abs(u), axis=-1, keepdims=True)
    return jnp.zeros((3, NS_PAD), f32).at[:, :NSAMP].set(surf.T).reshape(-1)


def _rsqrt(x):
    i = plsc.bitcast(x, i32)
    y = plsc.bitcast(jnp.int32(0x5F3759DF) - lax.shift_right_logical(i, 1), f32)
    for _ in range(3):
        y = y * (1.5 - 0.5 * x * y * y)
    return y


def _sqrt(x):
    return x * _rsqrt(x)


def _cross(ax, ay, az, bx, by, bz):
    return ay * bz - az * by, az * bx - ax * bz, ax * by - ay * bx


def _rotate(qw, ux, uy, uz, vx, vy, vz):
    tx, ty, tz = _cross(ux, uy, uz, vx, vy, vz)
    tx, ty, tz = 2.0 * tx, 2.0 * ty, 2.0 * tz
    cx, cy, cz = _cross(ux, uy, uz, tx, ty, tz)
    return vx + qw * tx + cx, vy + qw * ty + cy, vz + qw * tz + cz


_MESH = plsc.VectorSubcoreMesh(
    core_axis_name="c", subcore_axis_name="s", num_cores=2, num_subcores=16
)


@functools.partial(
    pl.kernel,
    out_type=jax.ShapeDtypeStruct((B, 2 * L), f32),
    mesh=_MESH,
    compiler_params=pltpu.CompilerParams(needs_layout_passes=False),
    scratch_types=[
        pltpu.VMEM((NCELL * 3,), f32),
        pltpu.VMEM((SG * 3,), f32),
        pltpu.VMEM((3 * SG_PAD,), f32),
        pltpu.VMEM((SG_PAD,), f32),
        pltpu.VMEM((3 * NS_PAD,), f32),
        pltpu.VMEM((P * 3,), f32),
        pltpu.VMEM((P * 3,), f32),
        pltpu.VMEM((P * 4,), f32),
        pltpu.VMEM((P,), i32),
        pltpu.VMEM((4 * L,), f32),
        pltpu.VMEM((L,), f32),
        pltpu.VMEM((2 * L,), f32),
        pltpu.SemaphoreType.DMA,
    ],
)
def _sc_kernel(shape_hbm, trans_hbm, quat_hbm, cp_hbm, pts_hbm, inuse_hbm,
               surf_hbm, out_hbm, cp_v, pts_raw, pts_t, tsdf_v, surf_v,
               shape_v, trans_v, quat_v, inuse_v, qn_v, acc2_v, out_v,
               cp_sem):
    b = lax.axis_index("s") * 2 + lax.axis_index("c")
    iota = jnp.arange(L, dtype=i32)

    cp_copy = pltpu.async_copy(cp_hbm.at[b], cp_v, cp_sem)

    pltpu.sync_copy(pts_hbm.at[b], pts_raw)
    pltpu.sync_copy(shape_hbm.at[b], shape_v)
    pltpu.sync_copy(trans_hbm.at[b], trans_v)
    pltpu.sync_copy(quat_hbm.at[b], quat_v)
    pltpu.sync_copy(inuse_hbm.at[b], inuse_v)
    pltpu.sync_copy(surf_hbm, surf_v)

    qw = plsc.load_gather(quat_v, [iota * 4])
    qx = plsc.load_gather(quat_v, [iota * 4 + 1])
    qy = plsc.load_gather(quat_v, [iota * 4 + 2])
    qz = plsc.load_gather(quat_v, [iota * 4 + 3])
    s = qw * qw + qx * qx + qy * qy + qz * qz
    n = s * _rsqrt(s)
    inv = 1.0 / (n + 1e-8)
    qn_v[pl.ds(0, L)] = qw * inv
    qn_v[pl.ds(L, L)] = qx * inv
    qn_v[pl.ds(2 * L, L)] = qy * inv
    qn_v[pl.ds(3 * L, L)] = qz * inv


    big_vec = jnp.full((L,), BIG, f32)

    def stage_body(ci, carry):
        for k in range(U1):
            base = (ci * U1 + k) * L
            ridx = jnp.minimum(base + iota, SG - 1) * 3
            for comp in range(3):
                v = plsc.load_gather(pts_raw, [ridx + comp])
                pts_t[pl.ds(comp * SG_PAD + base, L)] = v
            tsdf_v[pl.ds(base, L)] = big_vec
        return carry

    lax.fori_loop(0, N_CH1 // U1, stage_body, 0)

    def p1_body(p, carry):
        pvec = jnp.full((L,), 0, i32) + p
        iu = jnp.max(plsc.load_gather(inuse_v, [pvec]))

        @pl.when(iu > 0)
        def _():
            w = plsc.load_gather(qn_v, [pvec])
            nux = -plsc.load_gather(qn_v, [pvec + L])
            nuy = -plsc.load_gather(qn_v, [pvec + 2 * L])
            nuz = -plsc.load_gather(qn_v, [pvec + 3 * L])
            p3 = pvec * 3
            tx = plsc.load_gather(trans_v, [p3])
            ty = plsc.load_gather(trans_v, [p3 + 1])
            tz = plsc.load_gather(trans_v, [p3 + 2])
            sx = plsc.load_gather(shape_v, [p3])
            sy = plsc.load_gather(shape_v, [p3 + 1])
            sz = plsc.load_gather(shape_v, [p3 + 2])

            def body(ci, c2):
                for k in range(U1):
                    base = (ci * U1 + k) * L
                    vx = pts_t[pl.ds(base, L)] - tx
                    vy = pts_t[pl.ds(SG_PAD + base, L)] - ty
                    vz = pts_t[pl.ds(2 * SG_PAD + base, L)] - tz
                    lx, ly, lz = _rotate(w, nux, nuy, nuz, vx, vy, vz)
                    dx = jnp.maximum(jnp.abs(lx) - sx, 0.0)
                    dy = jnp.maximum(jnp.abs(ly) - sy, 0.0)
                    dz = jnp.maximum(jnp.abs(lz) - sz, 0.0)
                    t = dx * dx + dy * dy + dz * dz
                    tsdf_v[pl.ds(base, L)] = jnp.minimum(
                        tsdf_v[pl.ds(base, L)], t)
                return c2

            lax.fori_loop(0, N_CH1 // U1, body, 0)

        return carry

    lax.fori_loop(0, P, p1_body, 0)

    def red_body(ci, acc):
        for k in range(U1):
            base = (ci * U1 + k) * L
            v = tsdf_v[pl.ds(base, L)] + EPS
            sq = _sqrt(v)
            valid = (base + iota) < SG
            acc = acc + jnp.where(valid, sq, 0.0)
        return acc

    acc1 = lax.fori_loop(0, N_CH1 // U1, red_body, jnp.zeros((L,), f32))

    cp_copy.wait()
    acc2_v[:] = jnp.zeros((L,), f32)
    sqrt_eps = _sqrt(jnp.full((L,), EPS, f32))
    onehot0 = jnp.where(iota == 0, 1.0, 0.0).astype(f32)

    def p2_body(p, carry):
        pvec = jnp.full((L,), 0, i32) + p
        iu = jnp.max(plsc.load_gather(inuse_v, [pvec]))

        @pl.when(iu > 0)
        def _():
            w = plsc.load_gather(qn_v, [pvec])
            ux = plsc.load_gather(qn_v, [pvec + L])
            uy = plsc.load_gather(qn_v, [pvec + 2 * L])
            uz = plsc.load_gather(qn_v, [pvec + 3 * L])
            p3 = pvec * 3
            tx = plsc.load_gather(trans_v, [p3])
            ty = plsc.load_gather(trans_v, [p3 + 1])
            tz = plsc.load_gather(trans_v, [p3 + 2])
            sx = plsc.load_gather(shape_v, [p3])
            sy = plsc.load_gather(shape_v, [p3 + 1])
            sz = plsc.load_gather(shape_v, [p3 + 2])

            def body(ci, acc):
                for k in range(2):
                    base = (ci * 2 + k) * L
                    plx = surf_v[pl.ds(base, L)] * sx
                    ply = surf_v[pl.ds(NS_PAD + base, L)] * sy
                    plz = surf_v[pl.ds(2 * NS_PAD + base, L)] * sz
                    px, py, pz = _rotate(w, ux, uy, uz, plx, ply, plz)
                    px, py, pz = px + tx, py + ty, pz + tz
                    gx = jnp.clip(((px + 0.5) * float(GRID)).astype(i32),
                                  0, GRID - 1)
                    gy = jnp.clip(((py + 0.5) * float(GRID)).astype(i32),
                                  0, GRID - 1)
                    gz = jnp.clip(((pz + 0.5) * float(GRID)).astype(i32),
                                  0, GRID - 1)
                    lin = ((gx * GRID + gy) * GRID + gz) * 3
                    cx = plsc.load_gather(cp_v, [lin])
                    cy = plsc.load_gather(cp_v, [lin + 1])
                    cz = plsc.load_gather(cp_v, [lin + 2])
                    ex, ey, ez = px - cx, py - cy, pz - cz
                    d2 = ex * ex + ey * ey + ez * ez + EPS
                    dist = _sqrt(d2)
                    valid = (base + iota) < NSAMP
                    acc = acc + jnp.where(valid, dist, 0.0)
                return acc

            acc_p = lax.fori_loop(0, N_CH2 // 2, body, jnp.zeros((L,), f32))
            acc2_v[:] = acc2_v[:] + acc_p

        @pl.when(iu <= 0)
        def _():
            acc2_v[:] = acc2_v[:] + onehot0 * (float(NSAMP) * sqrt_eps)

        return carry

    lax.fori_loop(0, P, p2_body, 0)

    out_v[pl.ds(0, L)] = acc1
    out_v[pl.ds(L, L)] = acc2_v[:]
    pltpu.sync_copy(out_v, out_hbm.at[b])


def kernel(shape_rlt, trans_rlt, quat_rlt, CP, batchSamplepoint, inUse):
    out = _sc_kernel(
        shape_rlt.reshape(B, P * 3),
        trans_rlt.reshape(B, P * 3),
        quat_rlt.reshape(B, P * 4),
        CP.reshape(B, NCELL * 3),
        batchSamplepoint.reshape(B, SG * 3),
        inUse,
        _surf_table(),
    )
    discd2 = jnp.sum(out[:, :L]) / (B * SG)
    discd1 = jnp.sum(out[:, L:]) / (B * P * NSAMP)
    return discd1 + discd2


# speedup vs baseline: 19.7102x; 1.0186x over previous
import functools

import numpy as np

import jax
import jax.numpy as jnp
from jax import lax
from jax.experimental import pallas as pl
from jax.experimental.pallas import tpu as pltpu
from jax.experimental.pallas import tpu_sc as plsc

B = 32
P = 16
SG = 1000
NSAMP = 150
GRID = 32
EPS = 1e-12
BIG = 1e4

L = 16
SG_PAD = 1008
NS_PAD = 160
U1 = 3
N_CH1 = SG_PAD // L
N_CH2 = NS_PAD // L
NCELL = GRID * GRID * GRID

OFF_SHAPE = 0
OFF_TRANS = 48
OFF_QUAT = 96
OFF_IU = 160
OFF_PTS = 176
ROW = 3200

f32 = jnp.float32
i32 = jnp.int32


def _np_threefry_uniform(seed, n, lo, hi):
    rot = [np.uint32([13, 15, 26, 6]), np.uint32([17, 29, 16, 24])]

    def rotl(v, r):
        return ((v << np.uint32(r)) | (v >> np.uint32(32 - r))).astype(np.uint32)

    idx = np.arange(n, dtype=np.uint64)
    ks = [np.uint32(0), np.uint32(seed),
          np.uint32(np.uint32(0) ^ np.uint32(seed) ^ np.uint32(0x1BD11BDA))]
    x = [((idx >> np.uint64(32)).astype(np.uint32) + ks[0]).astype(np.uint32),
         ((idx & np.uint64(0xFFFFFFFF)).astype(np.uint32) + ks[1]).astype(np.uint32)]
    for i in range(5):
        for r in rot[i % 2]:
            x[0] = (x[0] + x[1]).astype(np.uint32)
            x[1] = rotl(x[1], r)
            x[1] = (x[1] ^ x[0]).astype(np.uint32)
        x[0] = (x[0] + ks[(i + 1) % 3]).astype(np.uint32)
        x[1] = (x[1] + ks[(i + 2) % 3] + np.uint32(i + 1)).astype(np.uint32)
    bits = x[0] ^ x[1]
    fb = (bits >> np.uint32(9)) | np.uint32(0x3F800000)
    f = fb.view(np.float32) - np.float32(1.0)
    out = f * np.float32(hi - lo) + np.float32(lo)
    return np.maximum(np.float32(lo), out)


def _surf_table():
    u = _np_threefry_uniform(42, NSAMP * 3, -1.0, 1.0).reshape(NSAMP, 3)
    surf = u / np.max(np.abs(u), axis=-1, keepdims=True)
    out = np.zeros((3, NS_PAD), np.float32)
    out[:, :NSAMP] = surf.T
    return out.reshape(-1)


_SURF_T = _surf_table()


def _rsqrt(x):
    i = plsc.bitcast(x, i32)
    y = plsc.bitcast(jnp.int32(0x5F3759DF) - lax.shift_right_logical(i, 1), f32)
    for _ in range(3):
        y = y * (1.5 - 0.5 * x * y * y)
    return y


def _sqrt(x):
    return x * _rsqrt(x)


def _cross(ax, ay, az, bx, by, bz):
    return ay * bz - az * by, az * bx - ax * bz, ax * by - ay * bx


def _rotate(qw, ux, uy, uz, vx, vy, vz):
    tx, ty, tz = _cross(ux, uy, uz, vx, vy, vz)
    tx, ty, tz = 2.0 * tx, 2.0 * ty, 2.0 * tz
    cx, cy, cz = _cross(ux, uy, uz, tx, ty, tz)
    return vx + qw * tx + cx, vy + qw * ty + cy, vz + qw * tz + cz


_MESH = plsc.VectorSubcoreMesh(
    core_axis_name="c", subcore_axis_name="s", num_cores=2, num_subcores=16
)


@functools.partial(
    pl.kernel,
    out_type=jax.ShapeDtypeStruct((B * 2 * L,), f32),
    mesh=_MESH,
    compiler_params=pltpu.CompilerParams(needs_layout_passes=False),
    scratch_types=[
        pltpu.VMEM((NCELL * 3,), f32),
        pltpu.VMEM((ROW,), f32),
        pltpu.VMEM((3 * SG_PAD,), f32),
        pltpu.VMEM((SG_PAD,), f32),
        pltpu.VMEM((3 * NS_PAD,), f32),
        pltpu.VMEM((4 * L,), f32),
        pltpu.VMEM((L,), f32),
        pltpu.VMEM((2 * L,), f32),
        pltpu.SemaphoreType.DMA,
    ],
)
def _sc_kernel(pack_hbm, cp_hbm, surf_hbm, out_hbm, cp_v, pack_v, pts_t,
               tsdf_v, surf_v, qn_v, acc2_v, out_v, cp_sem):
    b = lax.axis_index("s") * 2 + lax.axis_index("c")
    iota = jnp.arange(L, dtype=i32)

    cp_copy = pltpu.async_copy(cp_hbm.at[b], cp_v, cp_sem)

    pltpu.sync_copy(pack_hbm.at[b], pack_v)
    pltpu.sync_copy(surf_hbm, surf_v)

    qw = plsc.load_gather(pack_v, [iota * 4 + OFF_QUAT])
    qx = plsc.load_gather(pack_v, [iota * 4 + (OFF_QUAT + 1)])
    qy = plsc.load_gather(pack_v, [iota * 4 + (OFF_QUAT + 2)])
    qz = plsc.load_gather(pack_v, [iota * 4 + (OFF_QUAT + 3)])
    s = qw * qw + qx * qx + qy * qy + qz * qz
    n = s * _rsqrt(s)
    inv = 1.0 / (n + 1e-8)
    qn_v[pl.ds(0, L)] = qw * inv
    qn_v[pl.ds(L, L)] = qx * inv
    qn_v[pl.ds(2 * L, L)] = qy * inv
    qn_v[pl.ds(3 * L, L)] = qz * inv

    big_vec = jnp.full((L,), BIG, f32)

    def stage_body(ci, carry):
        for k in range(U1):
            base = (ci * U1 + k) * L
            ridx = jnp.minimum(base + iota, SG - 1) * 3 + OFF_PTS
            for comp in range(3):
                v = plsc.load_gather(pack_v, [ridx + comp])
                pts_t[pl.ds(comp * SG_PAD + base, L)] = v
            tsdf_v[pl.ds(base, L)] = big_vec
        return carry

    lax.fori_loop(0, N_CH1 // U1, stage_body, 0)

    def p1_body(p, carry):
        pvec = jnp.zeros((L,), i32) + p
        iu = jnp.max(plsc.load_gather(pack_v, [pvec + OFF_IU]))

        @pl.when(iu > 0.0)
        def _():
            w = plsc.load_gather(qn_v, [pvec])
            nux = -plsc.load_gather(qn_v, [pvec + L])
            nuy = -plsc.load_gather(qn_v, [pvec + 2 * L])
            nuz = -plsc.load_gather(qn_v, [pvec + 3 * L])
            p3 = pvec * 3
            tx = plsc.load_gather(pack_v, [p3 + OFF_TRANS])
            ty = plsc.load_gather(pack_v, [p3 + (OFF_TRANS + 1)])
            tz = plsc.load_gather(pack_v, [p3 + (OFF_TRANS + 2)])
            sx = plsc.load_gather(pack_v, [p3 + OFF_SHAPE])
            sy = plsc.load_gather(pack_v, [p3 + (OFF_SHAPE + 1)])
            sz = plsc.load_gather(pack_v, [p3 + (OFF_SHAPE + 2)])

            def body(ci, c2):
                for k in range(U1):
                    base = (ci * U1 + k) * L
                    vx = pts_t[pl.ds(base, L)] - tx
                    vy = pts_t[pl.ds(SG_PAD + base, L)] - ty
                    vz = pts_t[pl.ds(2 * SG_PAD + base, L)] - tz
                    lx, ly, lz = _rotate(w, nux, nuy, nuz, vx, vy, vz)
                    dx = jnp.maximum(jnp.abs(lx) - sx, 0.0)
                    dy = jnp.maximum(jnp.abs(ly) - sy, 0.0)
                    dz = jnp.maximum(jnp.abs(lz) - sz, 0.0)
                    t = dx * dx + dy * dy + dz * dz
                    tsdf_v[pl.ds(base, L)] = jnp.minimum(
                        tsdf_v[pl.ds(base, L)], t)
                return c2

            lax.fori_loop(0, N_CH1 // U1, body, 0)

        return carry

    lax.fori_loop(0, P, p1_body, 0)

    def red_body(ci, acc):
        for k in range(U1):
            base = (ci * U1 + k) * L
            v = tsdf_v[pl.ds(base, L)] + EPS
            sq = _sqrt(v)
            valid = (base + iota) < SG
            acc = acc + jnp.where(valid, sq, 0.0)
        return acc

    acc1 = lax.fori_loop(0, N_CH1 // U1, red_body, jnp.zeros((L,), f32))

    cp_copy.wait()
    acc2_v[:] = jnp.zeros((L,), f32)
    sqrt_eps = _sqrt(jnp.full((L,), EPS, f32))
    onehot0 = jnp.where(iota == 0, 1.0, 0.0).astype(f32)

    def p2_body(p, carry):
        pvec = jnp.zeros((L,), i32) + p
        iu = jnp.max(plsc.load_gather(pack_v, [pvec + OFF_IU]))

        @pl.when(iu > 0.0)
        def _():
            w = plsc.load_gather(qn_v, [pvec])
            ux = plsc.load_gather(qn_v, [pvec + L])
            uy = plsc.load_gather(qn_v, [pvec + 2 * L])
            uz = plsc.load_gather(qn_v, [pvec + 3 * L])
            p3 = pvec * 3
            tx = plsc.load_gather(pack_v, [p3 + OFF_TRANS])
            ty = plsc.load_gather(pack_v, [p3 + (OFF_TRANS + 1)])
            tz = plsc.load_gather(pack_v, [p3 + (OFF_TRANS + 2)])
            sx = plsc.load_gather(pack_v, [p3 + OFF_SHAPE])
            sy = plsc.load_gather(pack_v, [p3 + (OFF_SHAPE + 1)])
            sz = plsc.load_gather(pack_v, [p3 + (OFF_SHAPE + 2)])

            def body(ci, acc):
                for k in range(2):
                    base = (ci * 2 + k) * L
                    plx = surf_v[pl.ds(base, L)] * sx
                    ply = surf_v[pl.ds(NS_PAD + base, L)] * sy
                    plz = surf_v[pl.ds(2 * NS_PAD + base, L)] * sz
                    px, py, pz = _rotate(w, ux, uy, uz, plx, ply, plz)
                    px, py, pz = px + tx, py + ty, pz + tz
                    gx = jnp.clip(((px + 0.5) * float(GRID)).astype(i32),
                                  0, GRID - 1)
                    gy = jnp.clip(((py + 0.5) * float(GRID)).astype(i32),
                                  0, GRID - 1)
                    gz = jnp.clip(((pz + 0.5) * float(GRID)).astype(i32),
                                  0, GRID - 1)
                    lin = ((gx * GRID + gy) * GRID + gz) * 3
                    cx = plsc.load_gather(cp_v, [lin])
                    cy = plsc.load_gather(cp_v, [lin + 1])
                    cz = plsc.load_gather(cp_v, [lin + 2])
                    ex, ey, ez = px - cx, py - cy, pz - cz
                    d2 = ex * ex + ey * ey + ez * ez + EPS
                    dist = _sqrt(d2)
                    valid = (base + iota) < NSAMP
                    acc = acc + jnp.where(valid, dist, 0.0)
                return acc

            acc_p = lax.fori_loop(0, N_CH2 // 2, body, jnp.zeros((L,), f32))
            acc2_v[:] = acc2_v[:] + acc_p

        @pl.when(iu <= 0.0)
        def _():
            acc2_v[:] = acc2_v[:] + onehot0 * (float(NSAMP) * sqrt_eps)

        return carry

    lax.fori_loop(0, P, p2_body, 0)

    out_v[pl.ds(0, L)] = acc1 * (1.0 / (B * SG))
    out_v[pl.ds(L, L)] = acc2_v[:] * (1.0 / (B * P * NSAMP))
    pltpu.sync_copy(out_v, out_hbm.at[pl.ds(b * 2 * L, 2 * L)])


def kernel(shape_rlt, trans_rlt, quat_rlt, CP, batchSamplepoint, inUse):
    pack = jnp.concatenate(
        [
            shape_rlt.reshape(B, P * 3),
            trans_rlt.reshape(B, P * 3),
            quat_rlt.reshape(B, P * 4),
            inUse.astype(f32),
            batchSamplepoint.reshape(B, SG * 3),
            jnp.zeros((B, ROW - OFF_PTS - SG * 3), f32),
        ],
        axis=1,
    )
    out = _sc_kernel(pack, CP.reshape(B, NCELL * 3), jnp.asarray(_SURF_T))
    return jnp.sum(out)


# speedup vs baseline: 35.7917x; 1.8159x over previous
import functools

import numpy as np

import jax
import jax.numpy as jnp
from jax import lax
from jax.experimental import pallas as pl
from jax.experimental.pallas import tpu as pltpu
from jax.experimental.pallas import tpu_sc as plsc

B = 32
P = 16
SG = 1000
NSAMP = 150
GRID = 32
EPS = 1e-12
BIG = 1e4

L = 16
SG_PAD = 1008
NS_PAD = 160
U1 = 3
N_CH1 = SG_PAD // L
N_CH2 = NS_PAD // L
NCELL = GRID * GRID * GRID

OFF_SHAPE = 0
OFF_TRANS = 48
OFF_QUAT = 96
OFF_IU = 160
OFF_PTS = 176
ROW = 3200

f32 = jnp.float32
i32 = jnp.int32


def _np_threefry_uniform(seed, n, lo, hi):
    rot = [np.uint32([13, 15, 26, 6]), np.uint32([17, 29, 16, 24])]

    def rotl(v, r):
        return ((v << np.uint32(r)) | (v >> np.uint32(32 - r))).astype(np.uint32)

    idx = np.arange(n, dtype=np.uint64)
    ks = [np.uint32(0), np.uint32(seed),
          np.uint32(np.uint32(0) ^ np.uint32(seed) ^ np.uint32(0x1BD11BDA))]
    x = [((idx >> np.uint64(32)).astype(np.uint32) + ks[0]).astype(np.uint32),
         ((idx & np.uint64(0xFFFFFFFF)).astype(np.uint32) + ks[1]).astype(np.uint32)]
    for i in range(5):
        for r in rot[i % 2]:
            x[0] = (x[0] + x[1]).astype(np.uint32)
            x[1] = rotl(x[1], r)
            x[1] = (x[1] ^ x[0]).astype(np.uint32)
        x[0] = (x[0] + ks[(i + 1) % 3]).astype(np.uint32)
        x[1] = (x[1] + ks[(i + 2) % 3] + np.uint32(i + 1)).astype(np.uint32)
    bits = x[0] ^ x[1]
    fb = (bits >> np.uint32(9)) | np.uint32(0x3F800000)
    f = fb.view(np.float32) - np.float32(1.0)
    out = f * np.float32(hi - lo) + np.float32(lo)
    return np.maximum(np.float32(lo), out)


def _surf_table():
    u = _np_threefry_uniform(42, NSAMP * 3, -1.0, 1.0).reshape(NSAMP, 3)
    surf = u / np.max(np.abs(u), axis=-1, keepdims=True)
    out = np.zeros((3, NS_PAD), np.float32)
    out[:, :NSAMP] = surf.T
    return out.reshape(-1)


_SURF_T = _surf_table()


def _rsqrt(x):
    i = plsc.bitcast(x, i32)
    y = plsc.bitcast(jnp.int32(0x5F3759DF) - lax.shift_right_logical(i, 1), f32)
    for _ in range(3):
        y = y * (1.5 - 0.5 * x * y * y)
    return y


def _sqrt(x):
    return x * _rsqrt(x)


def _cross(ax, ay, az, bx, by, bz):
    return ay * bz - az * by, az * bx - ax * bz, ax * by - ay * bx


def _rotate(qw, ux, uy, uz, vx, vy, vz):
    tx, ty, tz = _cross(ux, uy, uz, vx, vy, vz)
    tx, ty, tz = 2.0 * tx, 2.0 * ty, 2.0 * tz
    cx, cy, cz = _cross(ux, uy, uz, tx, ty, tz)
    return vx + qw * tx + cx, vy + qw * ty + cy, vz + qw * tz + cz


_MESH = plsc.VectorSubcoreMesh(
    core_axis_name="c", subcore_axis_name="s", num_cores=2, num_subcores=16
)


@functools.partial(
    pl.kernel,
    out_type=jax.ShapeDtypeStruct((B * 2 * L,), f32),
    mesh=_MESH,
    compiler_params=pltpu.CompilerParams(needs_layout_passes=False),
    scratch_types=[
        pltpu.VMEM((NCELL * 3,), f32),
        pltpu.VMEM((ROW,), f32),
        pltpu.VMEM((SG_PAD,), f32),
        pltpu.VMEM((3 * NS_PAD,), f32),
        pltpu.VMEM((4 * L,), f32),
        pltpu.VMEM((L,), f32),
        pltpu.VMEM((2 * L,), f32),
        pltpu.SemaphoreType.DMA,
    ],
)
def _sc_kernel(pack_hbm, cp_hbm, surf_hbm, out_hbm, cp_v, pack_v, tsdf_v,
               surf_v, qn_v, acc2_v, out_v, cp_sem):
    b = lax.axis_index("s") * 2 + lax.axis_index("c")
    iota = jnp.arange(L, dtype=i32)

    cp_copy = pltpu.async_copy(cp_hbm.at[b], cp_v, cp_sem)

    pltpu.sync_copy(pack_hbm.at[b], pack_v)
    pltpu.sync_copy(surf_hbm, surf_v)

    qw = pack_v[pl.ds(OFF_QUAT, L)]
    qx = pack_v[pl.ds(OFF_QUAT + L, L)]
    qy = pack_v[pl.ds(OFF_QUAT + 2 * L, L)]
    qz = pack_v[pl.ds(OFF_QUAT + 3 * L, L)]
    s = qw * qw + qx * qx + qy * qy + qz * qz
    n = s * _rsqrt(s)
    inv = 1.0 / (n + 1e-8)
    qn_v[pl.ds(0, L)] = qw * inv
    qn_v[pl.ds(L, L)] = qx * inv
    qn_v[pl.ds(2 * L, L)] = qy * inv
    qn_v[pl.ds(3 * L, L)] = qz * inv

    big_vec = jnp.full((L,), BIG, f32)

    def init_body(ci, carry):
        for k in range(U1):
            tsdf_v[pl.ds((ci * U1 + k) * L, L)] = big_vec
        return carry

    lax.fori_loop(0, N_CH1 // U1, init_body, 0)

    def p1_body(p, carry):
        pvec = jnp.zeros((L,), i32) + p
        iu = jnp.max(plsc.load_gather(pack_v, [pvec + OFF_IU]))

        @pl.when(iu > 0.0)
        def _():
            w = plsc.load_gather(qn_v, [pvec])
            nux = -plsc.load_gather(qn_v, [pvec + L])
            nuy = -plsc.load_gather(qn_v, [pvec + 2 * L])
            nuz = -plsc.load_gather(qn_v, [pvec + 3 * L])
            tx = plsc.load_gather(pack_v, [pvec + OFF_TRANS])
            ty = plsc.load_gather(pack_v, [pvec + (OFF_TRANS + L)])
            tz = plsc.load_gather(pack_v, [pvec + (OFF_TRANS + 2 * L)])
            sx = plsc.load_gather(pack_v, [pvec + OFF_SHAPE])
            sy = plsc.load_gather(pack_v, [pvec + (OFF_SHAPE + L)])
            sz = plsc.load_gather(pack_v, [pvec + (OFF_SHAPE + 2 * L)])

            def body(ci, c2):
                for k in range(U1):
                    base = (ci * U1 + k) * L
                    vx = pack_v[pl.ds(OFF_PTS + base, L)] - tx
                    vy = pack_v[pl.ds(OFF_PTS + SG + base, L)] - ty
                    vz = pack_v[pl.ds(OFF_PTS + 2 * SG + base, L)] - tz
                    lx, ly, lz = _rotate(w, nux, nuy, nuz, vx, vy, vz)
                    dx = jnp.maximum(jnp.abs(lx) - sx, 0.0)
                    dy = jnp.maximum(jnp.abs(ly) - sy, 0.0)
                    dz = jnp.maximum(jnp.abs(lz) - sz, 0.0)
                    t = dx * dx + dy * dy + dz * dz
                    tsdf_v[pl.ds(base, L)] = jnp.minimum(
                        tsdf_v[pl.ds(base, L)], t)
                return c2

            lax.fori_loop(0, N_CH1 // U1, body, 0)

        return carry

    lax.fori_loop(0, P, p1_body, 0)

    def red_body(ci, acc):
        for k in range(U1):
            base = (ci * U1 + k) * L
            v = tsdf_v[pl.ds(base, L)] + EPS
            sq = _sqrt(v)
            valid = (base + iota) < SG
            acc = acc + jnp.where(valid, sq, 0.0)
        return acc

    acc1 = lax.fori_loop(0, N_CH1 // U1, red_body, jnp.zeros((L,), f32))

    cp_copy.wait()
    acc2_v[:] = jnp.zeros((L,), f32)
    sqrt_eps = _sqrt(jnp.full((L,), EPS, f32))
    onehot0 = jnp.where(iota == 0, 1.0, 0.0).astype(f32)

    def p2_body(p, carry):
        pvec = jnp.zeros((L,), i32) + p
        iu = jnp.max(plsc.load_gather(pack_v, [pvec + OFF_IU]))

        @pl.when(iu > 0.0)
        def _():
            w = plsc.load_gather(qn_v, [pvec])
            ux = plsc.load_gather(qn_v, [pvec + L])
            uy = plsc.load_gather(qn_v, [pvec + 2 * L])
            uz = plsc.load_gather(qn_v, [pvec + 3 * L])
            tx = plsc.load_gather(pack_v, [pvec + OFF_TRANS])
            ty = plsc.load_gather(pack_v, [pvec + (OFF_TRANS + L)])
            tz = plsc.load_gather(pack_v, [pvec + (OFF_TRANS + 2 * L)])
            sx = plsc.load_gather(pack_v, [pvec + OFF_SHAPE])
            sy = plsc.load_gather(pack_v, [pvec + (OFF_SHAPE + L)])
            sz = plsc.load_gather(pack_v, [pvec + (OFF_SHAPE + 2 * L)])

            def body(ci, acc):
                for k in range(2):
                    base = (ci * 2 + k) * L
                    plx = surf_v[pl.ds(base, L)] * sx
                    ply = surf_v[pl.ds(NS_PAD + base, L)] * sy
                    plz = surf_v[pl.ds(2 * NS_PAD + base, L)] * sz
                    px, py, pz = _rotate(w, ux, uy, uz, plx, ply, plz)
                    px, py, pz = px + tx, py + ty, pz + tz
                    gx = jnp.clip(((px + 0.5) * float(GRID)).astype(i32),
                                  0, GRID - 1)
                    gy = jnp.clip(((py + 0.5) * float(GRID)).astype(i32),
                                  0, GRID - 1)
                    gz = jnp.clip(((pz + 0.5) * float(GRID)).astype(i32),
                                  0, GRID - 1)
                    lin = gx * 3072 + gy * 32 + gz
                    cx = plsc.load_gather(cp_v, [lin])
                    cy = plsc.load_gather(cp_v, [lin + 1024])
                    cz = plsc.load_gather(cp_v, [lin + 2048])
                    ex, ey, ez = px - cx, py - cy, pz - cz
                    d2 = ex * ex + ey * ey + ez * ez + EPS
                    dist = _sqrt(d2)
                    valid = (base + iota) < NSAMP
                    acc = acc + jnp.where(valid, dist, 0.0)
                return acc

            acc_p = lax.fori_loop(0, N_CH2 // 2, body, jnp.zeros((L,), f32))
            acc2_v[:] = acc2_v[:] + acc_p

        @pl.when(iu <= 0.0)
        def _():
            acc2_v[:] = acc2_v[:] + onehot0 * (float(NSAMP) * sqrt_eps)

        return carry

    lax.fori_loop(0, P, p2_body, 0)

    out_v[pl.ds(0, L)] = acc1 * (1.0 / (B * SG))
    out_v[pl.ds(L, L)] = acc2_v[:] * (1.0 / (B * P * NSAMP))
    pltpu.sync_copy(out_v, out_hbm.at[pl.ds(b * 2 * L, 2 * L)])


def kernel(shape_rlt, trans_rlt, quat_rlt, CP, batchSamplepoint, inUse):
    pack = jnp.concatenate(
        [
            shape_rlt.transpose(0, 2, 1).reshape(B, 3 * P),
            trans_rlt.transpose(0, 2, 1).reshape(B, 3 * P),
            quat_rlt.transpose(0, 2, 1).reshape(B, 4 * P),
            inUse.astype(f32),
            batchSamplepoint.transpose(0, 2, 1).reshape(B, 3 * SG),
            jnp.zeros((B, ROW - OFF_PTS - 3 * SG), f32),
        ],
        axis=1,
    )
    cp = CP.transpose(0, 1, 4, 2, 3).reshape(B, NCELL * 3)
    out = _sc_kernel(pack, cp, jnp.asarray(_SURF_T))
    return jnp.sum(out)
